# Initial kernel scaffold; baseline (speedup 1.0000x reference)
import jax
import jax.numpy as jnp
from jax.experimental import pallas as pl


def kernel(x):
    raise NotImplementedError("write your pallas kernel here")


# speedup vs baseline: 1.0086x; 1.0086x over previous
import functools

import jax
import jax.numpy as jnp
from jax import lax
from jax.experimental import pallas as pl
from jax.experimental.pallas import tpu as pltpu
from jax.experimental.pallas import tpu_sc as plsc

_P = 0.5
_SCALE = 1.0 / (1.0 - _P)
_CHUNK = 10000
_LANES = 16


@functools.cache
def _keep_indices(rows: int, dim: int):
    keep = int(round(dim * (1.0 - _P)))
    noise = jax.random.uniform(jax.random.key(42), (rows, dim), dtype=jnp.float32)
    shuffle = jnp.argsort(noise, axis=-1)
    idx = jnp.sort(shuffle[:, :keep], axis=-1).astype(jnp.int32)
    return jax.device_get(idx)


@functools.cache
def _build(rows: int, dim: int, keep: int):
    info = plsc.get_sparse_core_info()
    nc, ns = info.num_cores, info.num_subcores
    nw = nc * ns
    rpw = rows // nw
    nchunks = keep // _CHUNK
    vecs = _CHUNK // _LANES

    mesh = plsc.VectorSubcoreMesh(
        core_axis_name="c", subcore_axis_name="s",
        num_cores=nc, num_subcores=ns)

    @functools.partial(
        pl.kernel,
        mesh=mesh,
        compiler_params=pltpu.CompilerParams(needs_layout_passes=False),
        out_type=jax.ShapeDtypeStruct((rows * keep,), jnp.float32),
        scratch_types=[
            pltpu.VMEM((dim,), jnp.float32),
            pltpu.VMEM((_CHUNK,), jnp.int32),
            pltpu.VMEM((_CHUNK,), jnp.float32),
        ],
    )
    def dimension_drop(x_hbm, idx_hbm, out_hbm, xrow, idxc, outc):
        wid = lax.axis_index("s") * nc + lax.axis_index("c")
        for r in range(rpw):
            row = wid * rpw + r
            xoff = pl.multiple_of(row * dim, 8)
            pltpu.sync_copy(x_hbm.at[pl.ds(xoff, dim)], xrow)
            for c in range(nchunks):
                koff = pl.multiple_of(row * keep + c * _CHUNK, 8)
                pltpu.sync_copy(idx_hbm.at[pl.ds(koff, _CHUNK)], idxc)

                def body(i, _):
                    base = i * _LANES
                    iv = idxc[pl.ds(base, _LANES)]
                    g = plsc.load_gather(xrow, [iv])
                    outc[pl.ds(base, _LANES)] = g * _SCALE
                    return 0

                lax.fori_loop(0, vecs, body, 0)
                pltpu.sync_copy(outc, out_hbm.at[pl.ds(koff, _CHUNK)])

    return dimension_drop


def kernel(x):
    rows, dim = x.shape
    keep = int(round(dim * (1.0 - _P)))
    idx = jnp.asarray(_keep_indices(rows, dim)).reshape(-1)
    out = _build(rows, dim, keep)(x.reshape(-1), idx)
    return out.reshape(rows, keep)


# speedup vs baseline: 37.9541x; 37.6321x over previous
import functools

import jax
import jax.numpy as jnp
from jax import lax
from jax.experimental import pallas as pl
from jax.experimental.pallas import tpu as pltpu
from jax.experimental.pallas import tpu_sc as plsc

_P = 0.5
_SCALE = 1.0 / (1.0 - _P)
_CHUNK = 10000
_LANES = 16


@functools.cache
def _keep_indices(rows: int, dim: int):
    keep = int(round(dim * (1.0 - _P)))
    with jax.ensure_compile_time_eval():
        noise = jax.random.uniform(
            jax.random.key(42), (rows, dim), dtype=jnp.float32)
        shuffle = jnp.argsort(noise, axis=-1)
        idx = jnp.sort(shuffle[:, :keep], axis=-1).astype(jnp.int32)
    return jax.device_get(idx)


@functools.cache
def _build(rows: int, dim: int, keep: int):
    info = plsc.get_sparse_core_info()
    nc, ns = info.num_cores, info.num_subcores
    nw = nc * ns
    rpw = rows // nw
    nchunks = keep // _CHUNK
    vecs = _CHUNK // _LANES

    mesh = plsc.VectorSubcoreMesh(
        core_axis_name="c", subcore_axis_name="s",
        num_cores=nc, num_subcores=ns)

    @functools.partial(
        pl.kernel,
        mesh=mesh,
        compiler_params=pltpu.CompilerParams(needs_layout_passes=False),
        out_type=jax.ShapeDtypeStruct((rows * keep,), jnp.float32),
        scratch_types=[
            pltpu.VMEM((dim,), jnp.float32),
            pltpu.VMEM((_CHUNK,), jnp.int32),
            pltpu.VMEM((_CHUNK,), jnp.float32),
        ],
    )
    def dimension_drop(x_hbm, idx_hbm, out_hbm, xrow, idxc, outc):
        wid = lax.axis_index("s") * nc + lax.axis_index("c")
        for r in range(rpw):
            row = wid * rpw + r
            xoff = pl.multiple_of(row * dim, 8)
            pltpu.sync_copy(x_hbm.at[pl.ds(xoff, dim)], xrow)
            for c in range(nchunks):
                koff = pl.multiple_of(row * keep + c * _CHUNK, 8)
                pltpu.sync_copy(idx_hbm.at[pl.ds(koff, _CHUNK)], idxc)

                def body(i, _):
                    base = i * _LANES
                    iv = idxc[pl.ds(base, _LANES)]
                    g = plsc.load_gather(xrow, [iv])
                    outc[pl.ds(base, _LANES)] = g * _SCALE
                    return 0

                lax.fori_loop(0, vecs, body, 0)
                pltpu.sync_copy(outc, out_hbm.at[pl.ds(koff, _CHUNK)])

    return dimension_drop


def kernel(x):
    rows, dim = x.shape
    keep = int(round(dim * (1.0 - _P)))
    idx = jnp.asarray(_keep_indices(rows, dim)).reshape(-1)
    out = _build(rows, dim, keep)(x.reshape(-1), idx)
    return out.reshape(rows, keep)


# speedup vs baseline: 45.7275x; 1.2048x over previous
import functools

import jax
import jax.numpy as jnp
from jax import lax
from jax.experimental import pallas as pl
from jax.experimental.pallas import tpu as pltpu
from jax.experimental.pallas import tpu_sc as plsc

_P = 0.5
_SCALE = 1.0 / (1.0 - _P)
_CHUNK = 10000
_LANES = 16


@functools.cache
def _keep_indices(rows: int, dim: int):
    keep = int(round(dim * (1.0 - _P)))
    cpu = jax.devices("cpu")[0]
    with jax.default_device(cpu), jax.ensure_compile_time_eval():
        noise = jax.random.uniform(
            jax.random.key(42), (rows, dim), dtype=jnp.float32)
        shuffle = jnp.argsort(noise, axis=-1)
        idx = jnp.sort(shuffle[:, :keep], axis=-1).astype(jnp.int32)
    return jax.device_get(idx)


@functools.cache
def _build(rows: int, dim: int, keep: int):
    info = plsc.get_sparse_core_info()
    nc, ns = info.num_cores, info.num_subcores
    nw = nc * ns
    rpw = rows // nw
    nchunks = keep // _CHUNK
    vecs = _CHUNK // _LANES

    mesh = plsc.VectorSubcoreMesh(
        core_axis_name="c", subcore_axis_name="s",
        num_cores=nc, num_subcores=ns)

    @functools.partial(
        pl.kernel,
        mesh=mesh,
        compiler_params=pltpu.CompilerParams(needs_layout_passes=False),
        out_type=jax.ShapeDtypeStruct((rows * keep,), jnp.float32),
        scratch_types=[
            pltpu.VMEM((dim,), jnp.float32),
            pltpu.VMEM((_CHUNK,), jnp.int32),
            pltpu.VMEM((_CHUNK,), jnp.float32),
        ],
    )
    def dimension_drop(x_hbm, idx_hbm, out_hbm, xrow, idxc, outc):
        wid = lax.axis_index("s") * nc + lax.axis_index("c")
        for r in range(rpw):
            row = wid * rpw + r
            xoff = pl.multiple_of(row * dim, 8)
            pltpu.sync_copy(x_hbm.at[pl.ds(xoff, dim)], xrow)
            for c in range(nchunks):
                koff = pl.multiple_of(row * keep + c * _CHUNK, 8)
                pltpu.sync_copy(idx_hbm.at[pl.ds(koff, _CHUNK)], idxc)

                @plsc.parallel_loop(0, _CHUNK, _LANES, unroll=8)
                def body(i):
                    iv = idxc[pl.ds(i, _LANES)]
                    g = plsc.load_gather(xrow, [iv])
                    outc[pl.ds(i, _LANES)] = g * _SCALE
                pltpu.sync_copy(outc, out_hbm.at[pl.ds(koff, _CHUNK)])

    return dimension_drop


def kernel(x):
    rows, dim = x.shape
    keep = int(round(dim * (1.0 - _P)))
    idx = jnp.asarray(_keep_indices(rows, dim)).reshape(-1)
    out = _build(rows, dim, keep)(x.reshape(-1), idx)
    return out.reshape(rows, keep)


# speedup vs baseline: 50.6485x; 1.1076x over previous
import functools

import jax
import jax.numpy as jnp
import numpy as np
from jax import lax
from jax.experimental import pallas as pl
from jax.experimental.pallas import tpu as pltpu
from jax.experimental.pallas import tpu_sc as plsc

_P = 0.5
_SCALE = 1.0 / (1.0 - _P)
_CHUNK = 10000
_LANES = 16


@functools.cache
def _keep_indices(rows: int, dim: int):
    keep = int(round(dim * (1.0 - _P)))
    cpu = jax.devices("cpu")[0]
    with jax.default_device(cpu), jax.ensure_compile_time_eval():
        noise = jax.random.uniform(
            jax.random.key(42), (rows, dim), dtype=jnp.float32)
        shuffle = jnp.argsort(noise, axis=-1)
        idx = jnp.sort(shuffle[:, :keep], axis=-1).astype(jnp.int32)
    return jax.device_get(idx)


@functools.cache
def _plan(rows: int, dim: int):
    idx = _keep_indices(rows, dim)
    keep = idx.shape[1]
    nchunks = keep // _CHUNK
    mins = [int(idx[:, c * _CHUNK].min()) for c in range(nchunks)]
    maxs = [int(idx[:, (c + 1) * _CHUNK - 1].max()) for c in range(nchunks)]
    wlen = max(mx - (mn & ~7) + 1 for mn, mx in zip(mins, maxs))
    wlen = (wlen + 7) & ~7
    wstarts = []
    for mn, mx in zip(mins, maxs):
        w = min(mn & ~7, dim - wlen)
        assert w >= 0 and mx < w + wlen
        wstarts.append(w)
    reb = idx.astype(np.int64).copy()
    for c, w in enumerate(wstarts):
        reb[:, c * _CHUNK:(c + 1) * _CHUNK] -= w
    return reb.astype(np.int32).reshape(-1), tuple(wstarts), wlen


@functools.cache
def _build(rows: int, dim: int, keep: int, wlen: int, wstarts: tuple):
    info = plsc.get_sparse_core_info()
    nc, ns = info.num_cores, info.num_subcores
    nw = nc * ns
    rpw = rows // nw
    nchunks = keep // _CHUNK
    ntasks = rpw * nchunks

    mesh = plsc.VectorSubcoreMesh(
        core_axis_name="c", subcore_axis_name="s",
        num_cores=nc, num_subcores=ns)

    @functools.partial(
        pl.kernel,
        mesh=mesh,
        compiler_params=pltpu.CompilerParams(needs_layout_passes=False),
        out_type=jax.ShapeDtypeStruct((rows * keep,), jnp.float32),
        scratch_types=[
            pltpu.VMEM((wlen,), jnp.float32),
            pltpu.VMEM((wlen,), jnp.float32),
            pltpu.VMEM((_CHUNK,), jnp.int32),
            pltpu.VMEM((_CHUNK,), jnp.int32),
            pltpu.VMEM((_CHUNK,), jnp.float32),
            pltpu.VMEM((_CHUNK,), jnp.float32),
            pltpu.SemaphoreType.DMA,
            pltpu.SemaphoreType.DMA,
            pltpu.SemaphoreType.DMA,
            pltpu.SemaphoreType.DMA,
        ],
    )
    def dimension_drop(x_hbm, idx_hbm, out_hbm, xw0, xw1, idb0, idb1,
                       ob0, ob1, sin0, sin1, sout0, sout1):
        wid = lax.axis_index("s") * nc + lax.axis_index("c")
        xw = (xw0, xw1)
        idb = (idb0, idb1)
        ob = (ob0, ob1)
        sin = (sin0, sin1)
        sout = (sout0, sout1)

        def offsets(t):
            r, c = divmod(t, nchunks)
            row = wid * rpw + r
            xoff = pl.multiple_of(row * dim + wstarts[c], 8)
            koff = pl.multiple_of(row * keep + c * _CHUNK, 8)
            return xoff, koff

        def in_copies(t, b):
            xoff, koff = offsets(t)
            return (
                pltpu.make_async_copy(
                    x_hbm.at[pl.ds(xoff, wlen)], xw[b], sin[b]),
                pltpu.make_async_copy(
                    idx_hbm.at[pl.ds(koff, _CHUNK)], idb[b], sin[b]),
            )

        def out_copy(t, b):
            _, koff = offsets(t)
            return pltpu.make_async_copy(
                ob[b], out_hbm.at[pl.ds(koff, _CHUNK)], sout[b])

        for cp in in_copies(0, 0):
            cp.start()
        for t in range(ntasks):
            b = t & 1
            if t + 1 < ntasks:
                for cp in in_copies(t + 1, 1 - b):
                    cp.start()
            for cp in in_copies(t, b):
                cp.wait()
            if t >= 2:
                out_copy(t - 2, b).wait()

            @plsc.parallel_loop(0, _CHUNK, _LANES, unroll=5)
            def body(i):
                iv = idb[b][pl.ds(i, _LANES)]
                g = plsc.load_gather(xw[b], [iv])
                ob[b][pl.ds(i, _LANES)] = g * _SCALE

            out_copy(t, b).start()
        out_copy(ntasks - 2, 0).wait()
        out_copy(ntasks - 1, 1).wait()

    return dimension_drop


def kernel(x):
    rows, dim = x.shape
    keep = int(round(dim * (1.0 - _P)))
    reb, wstarts, wlen = _plan(rows, dim)
    fn = _build(rows, dim, keep, wlen, wstarts)
    out = fn(x.reshape(-1), jnp.asarray(reb))
    return out.reshape(rows, keep)


# speedup vs baseline: 77.9523x; 1.5391x over previous
import functools

import jax
import jax.numpy as jnp
import numpy as np
from jax import lax
from jax.experimental import pallas as pl
from jax.experimental.pallas import tpu as pltpu
from jax.experimental.pallas import tpu_sc as plsc

_P = 0.5
_SCALE = 1.0 / (1.0 - _P)
_CH = 1024
_BR = 8
_LANES = 16


@functools.cache
def _keep_indices(rows: int, dim: int):
    keep = int(round(dim * (1.0 - _P)))
    cpu = jax.devices("cpu")[0]
    with jax.default_device(cpu), jax.ensure_compile_time_eval():
        noise = jax.random.uniform(
            jax.random.key(42), (rows, dim), dtype=jnp.float32)
        shuffle = jnp.argsort(noise, axis=-1)
        idx = jnp.sort(shuffle[:, :keep], axis=-1).astype(jnp.int32)
    return jax.device_get(idx)


@functools.cache
def _plan(rows: int, dim: int):
    idx = _keep_indices(rows, dim)
    keep = idx.shape[1]
    nblocks = rows // _BR
    nchunks = -(-keep // _CH)
    wstarts, wends = [], []
    for c in range(nchunks):
        lo = c * _CH
        hi = min(lo + _CH, keep)
        mn = int(idx[:, lo].min()) & ~127
        mx = int(idx[:, hi - 1].max())
        wstarts.append(mn)
        wends.append(mx + 1)
    wlen = max(e - s for s, e in zip(wstarts, wends))
    wlen = (wlen + 127) & ~127
    wstart_tail = (dim - wlen) & ~127
    wlen_tail = dim - wstart_tail
    tail = []
    for c in range(nchunks):
        if wends[c] > wstarts[c] + wlen or wstarts[c] + wlen > dim:
            wstarts[c] = wstart_tail
            tail.append(c)
        assert wstarts[c] >= 0
        cap = wlen_tail if c in tail else wlen
        assert wends[c] <= wstarts[c] + cap
    assert tail in ([], [nchunks - 1])
    reb = np.zeros((nblocks, nchunks, _BR, _CH), np.int32)
    for c in range(nchunks):
        lo = c * _CH
        hi = min(lo + _CH, keep)
        for b in range(nblocks):
            sl = idx[b * _BR:(b + 1) * _BR, lo:hi] - wstarts[c]
            reb[b, c, :, :hi - lo] = sl
    return reb.reshape(-1), tuple(wstarts), wlen, wlen_tail, nchunks


@functools.cache
def _build(rows: int, dim: int, keep: int, wlen: int, wlen_tail: int,
           wstarts: tuple, nchunks: int):
    info = plsc.get_sparse_core_info()
    nc, ns = info.num_cores, info.num_subcores
    nw = nc * ns
    nblocks = rows // _BR
    assert nblocks * 2 == nw
    ktasks = (nchunks + 1) // 2
    last_len = keep - (nchunks - 1) * _CH

    mesh = plsc.VectorSubcoreMesh(
        core_axis_name="c", subcore_axis_name="s",
        num_cores=nc, num_subcores=ns)

    @functools.partial(
        pl.kernel,
        mesh=mesh,
        compiler_params=pltpu.CompilerParams(needs_layout_passes=False),
        out_type=jax.ShapeDtypeStruct((rows, keep), jnp.float32),
        scratch_types=[
            pltpu.VMEM((_BR, wlen_tail), jnp.float32),
            pltpu.VMEM((_BR, wlen_tail), jnp.float32),
            pltpu.VMEM((_BR * _CH,), jnp.int32),
            pltpu.VMEM((_BR * _CH,), jnp.int32),
            pltpu.VMEM((_BR, _CH), jnp.float32),
            pltpu.VMEM((_BR, _CH), jnp.float32),
            pltpu.VMEM((_BR, last_len), jnp.float32),
            pltpu.SemaphoreType.DMA,
            pltpu.SemaphoreType.DMA,
            pltpu.SemaphoreType.DMA,
            pltpu.SemaphoreType.DMA,
        ],
    )
    def dimension_drop(x_hbm, idx_hbm, out_hbm, xw0, xw1, idb0, idb1,
                       ob0, ob1, obl, sin0, sin1, sout0, sout1):
        wid = lax.axis_index("s") * nc + lax.axis_index("c")
        blk = wid // 2
        par = wid - 2 * blk
        r8 = pl.multiple_of(blk * _BR, _BR)
        xw = (xw0, xw1)
        idb = (idb0, idb1)
        ob = (ob0, ob1)
        sin = (sin0, sin1)
        sout = (sout0, sout1)

        even = [wstarts[min(2 * k, nchunks - 1)] for k in range(ktasks)]
        odd = [wstarts[min(2 * k + 1, nchunks - 1)] for k in range(ktasks)]

        def chunk_id(k):
            return 2 * k + par

        def in_copies(k, b):
            koff = pl.multiple_of(
                (blk * nchunks + chunk_id(k)) * (_BR * _CH), 8)
            if 2 * k == nchunks - 1:
                xcp = pltpu.make_async_copy(
                    x_hbm.at[pl.ds(r8, _BR),
                             pl.ds(wstarts[nchunks - 1], wlen_tail)],
                    xw[b], sin[b])
            else:
                ws = pl.multiple_of(
                    jnp.where(par == 0, even[k], odd[k]).astype(jnp.int32),
                    128)
                xcp = pltpu.make_async_copy(
                    x_hbm.at[pl.ds(r8, _BR), pl.ds(ws, wlen)],
                    xw[b].at[pl.ds(0, _BR), pl.ds(0, wlen)], sin[b])
            return (
                xcp,
                pltpu.make_async_copy(
                    idx_hbm.at[pl.ds(koff, _BR * _CH)], idb[b], sin[b]),
            )

        def out_copy(k, b, clen):
            if clen == _CH:
                coff = pl.multiple_of(chunk_id(k) * _CH, 128)
                src = ob[b]
            else:
                coff = (nchunks - 1) * _CH
                src = obl
            return pltpu.make_async_copy(
                src, out_hbm.at[pl.ds(r8, _BR), pl.ds(coff, clen)], sout[b])

        def task_live(k):
            return chunk_id(k) < nchunks

        def clen_of(k, p):
            c = 2 * k + p
            if c == nchunks - 1:
                return last_len
            return _CH

        for cp in in_copies(0, 0):
            cp.start()
        for k in range(ktasks):
            b = k & 1
            if k + 1 < ktasks:

                @pl.when(task_live(k + 1))
                def _():
                    for cp in in_copies(k + 1, 1 - b):
                        cp.start()

            @pl.when(task_live(k))
            def _():
                for cp in in_copies(k, b):
                    cp.wait()
                if k >= 2:
                    out_copy(k - 2, b, _CH).wait()

                if 2 * k == nchunks - 1:
                    @plsc.parallel_loop(0, _BR * _CH, _LANES, unroll=4)
                    def body(i):
                        row = i // _CH
                        rv = jnp.zeros((_LANES,), jnp.int32) + row
                        cv = idb[b][pl.ds(i, _LANES)]
                        g = plsc.load_gather(xw[b], [rv, cv]) * _SCALE
                        cst = lax.iota(jnp.int32, _LANES) + (i - row * _CH)
                        plsc.store_scatter(obl, [rv, cst], g,
                                           mask=cst < last_len)
                else:

                    @plsc.parallel_loop(0, _BR * _CH, _LANES, unroll=4)
                    def body(i):
                        row = i // _CH
                        rv = jnp.zeros((_LANES,), jnp.int32) + row
                        cv = idb[b][pl.ds(i, _LANES)]
                        g = plsc.load_gather(xw[b], [rv, cv]) * _SCALE
                        cst = lax.iota(jnp.int32, _LANES) + (i - row * _CH)
                        plsc.store_scatter(ob[b], [rv, cst], g)

                if 2 * k == nchunks - 1:
                    out_copy(k, b, last_len).start()
                elif 2 * k + 1 == nchunks - 1:

                    @pl.when(par == 1)
                    def _():
                        out_copy(k, b, last_len).start()

                    @pl.when(par == 0)
                    def _():
                        out_copy(k, b, _CH).start()
                else:
                    out_copy(k, b, _CH).start()

        for k in range(max(0, ktasks - 3), ktasks):

            @pl.when(task_live(k) & jnp.logical_not(task_live(k + 2)))
            def _():
                out_copy(k, k & 1, clen_of(k, 0)).wait()

    return dimension_drop


def kernel(x):
    rows, dim = x.shape
    keep = int(round(dim * (1.0 - _P)))
    reb, wstarts, wlen, wlen_tail, nchunks = _plan(rows, dim)
    fn = _build(rows, dim, keep, wlen, wlen_tail, wstarts, nchunks)
    return fn(x, jnp.asarray(reb))
